# Initial kernel scaffold; baseline (speedup 1.0000x reference)
import jax
import jax.numpy as jnp
from jax.experimental import pallas as pl


def kernel(x, edge_index, W1, b1, W2, b2):
    raise NotImplementedError("write your pallas kernel here")


# speedup vs baseline: 12.1034x; 12.1034x over previous
import functools

import jax
import jax.numpy as jnp
from jax import lax
from jax.experimental import pallas as pl
from jax.experimental.pallas import tpu as pltpu
from jax.experimental.pallas import tpu_sc as plsc

N = 10000
NPAD = 10240
E = 320000
DIN = 128
DOUT = 64
K_HOPS = 3

NC = 2
NS = 16
NW = NC * NS
CH = 128
CPW = 80
EPAD = NW * CPW * CH
ROWS_PER_TILE = NPAD // NS

_mesh = plsc.VectorSubcoreMesh(core_axis_name="c", subcore_axis_name="s")
_sc_params = pltpu.CompilerParams(use_tc_tiling_on_sc=False)


@functools.partial(
    pl.kernel,
    out_type=jax.ShapeDtypeStruct((NC * NPAD, 16), jnp.float32),
    mesh=_mesh,
    scratch_types=[
        pltpu.VMEM((CPW, CH), jnp.int32),
        pltpu.VMEM((CH, 16), jnp.float32),
        pltpu.VMEM((ROWS_PER_TILE, 16), jnp.float32),
        pltpu.VMEM_SHARED((NPAD, 16), jnp.float32),
    ],
    compiler_params=_sc_params,
)
def _sc_degree(dst_hbm, out_hbm, dst_v, obuf, zbuf, acc_sp):
    cid = lax.axis_index("c")
    sid = lax.axis_index("s")
    wid = cid * NS + sid
    pltpu.sync_copy(dst_hbm.at[wid], dst_v)
    one_hot = jnp.where(lax.iota(jnp.int32, 16) == 0,
                        jnp.float32(1.0), jnp.float32(0.0))
    zeros16 = jnp.zeros((16,), jnp.float32)

    def fill_obuf(r, _):
        obuf[r, :] = one_hot
        return 0

    lax.fori_loop(0, CH, fill_obuf, 0)

    def fill_z(r, _):
        zbuf[r, :] = zeros16
        return 0

    lax.fori_loop(0, ROWS_PER_TILE, fill_z, 0)
    pltpu.sync_copy(zbuf, acc_sp.at[pl.ds(sid * ROWS_PER_TILE, ROWS_PER_TILE)])
    plsc.subcore_barrier()

    def chunk(ci, _):
        pltpu.sync_copy(obuf, acc_sp.at[dst_v.at[ci]], add=True)
        return 0

    lax.fori_loop(0, CPW, chunk, 0)
    plsc.subcore_barrier()
    base = cid * NPAD + sid * ROWS_PER_TILE
    pltpu.sync_copy(acc_sp.at[pl.ds(sid * ROWS_PER_TILE, ROWS_PER_TILE)],
                    out_hbm.at[pl.ds(base, ROWS_PER_TILE)])


@functools.partial(
    pl.kernel,
    out_type=jax.ShapeDtypeStruct((NC * NPAD, DOUT), jnp.float32),
    mesh=_mesh,
    scratch_types=[
        pltpu.VMEM((CPW, CH), jnp.int32),
        pltpu.VMEM((CPW, CH), jnp.int32),
        pltpu.VMEM((2, CH, DOUT), jnp.float32),
        pltpu.VMEM((ROWS_PER_TILE, DOUT), jnp.float32),
        pltpu.VMEM_SHARED((NPAD, DOUT), jnp.float32),
        pltpu.SemaphoreType.DMA,
        pltpu.SemaphoreType.DMA,
    ],
    compiler_params=_sc_params,
)
def _sc_hop(h_hbm, src_hbm, dst_hbm, out_hbm,
            src_v, dst_v, gbuf, zbuf, acc_sp, sem0, sem1):
    cid = lax.axis_index("c")
    sid = lax.axis_index("s")
    wid = cid * NS + sid
    pltpu.sync_copy(src_hbm.at[wid], src_v)
    pltpu.sync_copy(dst_hbm.at[wid], dst_v)
    zeros16 = jnp.zeros((16,), jnp.float32)

    def fill_z(r, _):
        for j in range(DOUT // 16):
            zbuf[r, pl.ds(j * 16, 16)] = zeros16
        return 0

    lax.fori_loop(0, ROWS_PER_TILE, fill_z, 0)
    pltpu.sync_copy(zbuf, acc_sp.at[pl.ds(sid * ROWS_PER_TILE, ROWS_PER_TILE)])
    plsc.subcore_barrier()

    sems = (sem0, sem1)
    pltpu.async_copy(h_hbm.at[src_v.at[0]], gbuf.at[0], sem0)
    pltpu.async_copy(h_hbm.at[src_v.at[1]], gbuf.at[1], sem1)

    def outer(j, _):
        for b in range(2):
            ci = j * 2 + b
            pltpu.make_async_copy(h_hbm.at[src_v.at[ci]], gbuf.at[b],
                                  sems[b]).wait()
            pltpu.sync_copy(gbuf.at[b], acc_sp.at[dst_v.at[ci]], add=True)

            @pl.when(ci + 2 < CPW)
            def _():
                pltpu.async_copy(h_hbm.at[src_v.at[ci + 2]], gbuf.at[b],
                                 sems[b])
        return 0

    lax.fori_loop(0, CPW // 2, outer, 0)
    plsc.subcore_barrier()
    base = cid * NPAD + sid * ROWS_PER_TILE
    pltpu.sync_copy(acc_sp.at[pl.ds(sid * ROWS_PER_TILE, ROWS_PER_TILE)],
                    out_hbm.at[pl.ds(base, ROWS_PER_TILE)])


def _tc_w_body(w1_ref, w2_ref, b1_ref, b2_ref, w_ref, bv_ref):
    w_ref[...] = jnp.dot(w1_ref[...], w2_ref[...],
                         preferred_element_type=jnp.float32)
    bv_ref[...] = jnp.dot(b1_ref[...], w2_ref[...],
                          preferred_element_type=jnp.float32) + b2_ref[...]


_tc_w = pl.pallas_call(
    _tc_w_body,
    out_shape=(
        jax.ShapeDtypeStruct((DIN, DOUT), jnp.float32),
        jax.ShapeDtypeStruct((8, DOUT), jnp.float32),
    ),
)


def _tc_h0_body(x_ref, w_ref, dp_ref, o_ref):
    deg = dp_ref[0, :, 0] + dp_ref[1, :, 0] + 1.0
    s = lax.rsqrt(deg)
    o_ref[...] = jnp.dot(x_ref[...], w_ref[...],
                         preferred_element_type=jnp.float32) * s[:, None]


_tc_h0 = pl.pallas_call(
    _tc_h0_body,
    grid=(NPAD // 128,),
    in_specs=[
        pl.BlockSpec((128, DIN), lambda i: (i, 0)),
        pl.BlockSpec((DIN, DOUT), lambda i: (0, 0)),
        pl.BlockSpec((2, 128, 16), lambda i: (0, i, 0)),
    ],
    out_specs=pl.BlockSpec((128, DOUT), lambda i: (i, 0)),
    out_shape=jax.ShapeDtypeStruct((NPAD, DOUT), jnp.float32),
)


def _tc_comb_body(pp_ref, h_ref, dp_ref, o_ref):
    deg = dp_ref[0, :, 0] + dp_ref[1, :, 0] + 1.0
    d = 1.0 / deg
    o_ref[...] = (pp_ref[0] + pp_ref[1] + h_ref[...]) * d[:, None]


_tc_comb = pl.pallas_call(
    _tc_comb_body,
    grid=(NPAD // 128,),
    in_specs=[
        pl.BlockSpec((2, 128, DOUT), lambda i: (0, i, 0)),
        pl.BlockSpec((128, DOUT), lambda i: (i, 0)),
        pl.BlockSpec((2, 128, 16), lambda i: (0, i, 0)),
    ],
    out_specs=pl.BlockSpec((128, DOUT), lambda i: (i, 0)),
    out_shape=jax.ShapeDtypeStruct((NPAD, DOUT), jnp.float32),
)


def _tc_final_body(pp_ref, h_ref, dp_ref, bv_ref, o_ref):
    deg = dp_ref[0, :, 0] + dp_ref[1, :, 0] + 1.0
    s = lax.rsqrt(deg)
    o_ref[...] = ((pp_ref[0] + pp_ref[1] + h_ref[...]) * s[:, None]
                  + bv_ref[0:1, :])


_tc_final = pl.pallas_call(
    _tc_final_body,
    grid=(NPAD // 128,),
    in_specs=[
        pl.BlockSpec((2, 128, DOUT), lambda i: (0, i, 0)),
        pl.BlockSpec((128, DOUT), lambda i: (i, 0)),
        pl.BlockSpec((2, 128, 16), lambda i: (0, i, 0)),
        pl.BlockSpec((8, DOUT), lambda i: (0, 0)),
    ],
    out_specs=pl.BlockSpec((128, DOUT), lambda i: (i, 0)),
    out_shape=jax.ShapeDtypeStruct((NPAD, DOUT), jnp.float32),
)


@jax.jit
def kernel(x, edge_index, W1, b1, W2, b2):
    src = edge_index[0].astype(jnp.int32)
    dst = edge_index[1].astype(jnp.int32)
    pad = jnp.full((EPAD - E,), N, dtype=jnp.int32)
    srcr = jnp.concatenate([src, pad]).reshape(NW, CPW, CH)
    dstr = jnp.concatenate([dst, pad]).reshape(NW, CPW, CH)
    xp = jnp.pad(x, ((0, NPAD - N), (0, 0)))
    b1r = jnp.broadcast_to(b1[None, :], (8, DIN))
    b2r = jnp.broadcast_to(b2[None, :], (8, DOUT))

    degp = _sc_degree(dstr).reshape(NC, NPAD, 16)
    w_f, bv = _tc_w(W1, W2, b1r, b2r)
    h = _tc_h0(xp, w_f, degp)
    for hop in range(K_HOPS):
        pp = _sc_hop(h, srcr, dstr).reshape(NC, NPAD, DOUT)
        if hop < K_HOPS - 1:
            h = _tc_comb(pp, h, degp)
        else:
            h = _tc_final(pp, h, degp, bv)
    return h[:N]
